# baseline (speedup 1.0000x reference)
import jax
import jax.numpy as jnp
from jax.experimental import pallas as pl
from jax.experimental.pallas import tpu as pltpu

_EPS = 1e-08
_BLOCK = 1024


def _route_block(hb, a_n):
    h_n = hb / jnp.maximum(
        jnp.sqrt(jnp.sum(hb * hb, axis=1, keepdims=True)), _EPS)
    scores = jax.lax.dot_general(
        h_n, a_n, (((1,), (1,)), ((), ())),
        preferred_element_type=jnp.float32)
    return jnp.argmax(scores, axis=1).astype(jnp.int32)


_NSPLIT = 4
_CHUNK = _BLOCK // _NSPLIT


def _router_kernel(h_ref, a_ref, out_ref, buf, sems):
    nblk = h_ref.shape[0] // _BLOCK

    def copies(i, slot):
        return [
            pltpu.make_async_copy(
                h_ref.at[pl.ds(i * _BLOCK + c * _CHUNK, _CHUNK), :],
                buf.at[slot, pl.ds(c * _CHUNK, _CHUNK), :],
                sems.at[slot, c])
            for c in range(_NSPLIT)
        ]

    a = a_ref[...]
    a_n = a / jnp.maximum(
        jnp.sqrt(jnp.sum(a * a, axis=1, keepdims=True)), _EPS)

    for cp in copies(0, 0):
        cp.start()
    for i in range(nblk):
        slot = i % 2
        if i + 1 < nblk:
            for cp in copies(i + 1, 1 - slot):
                cp.start()
        for cp in copies(i, slot):
            cp.wait()
        out_ref[pl.ds(i * _BLOCK, _BLOCK), :] = (
            _route_block(buf[slot], a_n)[:, None])


def kernel(h, A):
    B, D = h.shape
    K = A.shape[0]
    return pl.pallas_call(
        _router_kernel,
        in_specs=[
            pl.BlockSpec(memory_space=pltpu.HBM),
            pl.BlockSpec((K, D), lambda: (0, 0)),
        ],
        out_specs=pl.BlockSpec((B, 1), lambda: (0, 0)),
        out_shape=jax.ShapeDtypeStruct((B, 1), jnp.int32),
        scratch_shapes=[
            pltpu.VMEM((2, _BLOCK, D), jnp.float32),
            pltpu.SemaphoreType.DMA((2, 4)),
        ],
        compiler_params=pltpu.CompilerParams(
            vmem_limit_bytes=100 * 1024 * 1024,
        ),
    )(h, A)


# speedup vs baseline: 1.3944x; 1.3944x over previous
import jax
import jax.numpy as jnp
from jax.experimental import pallas as pl
from jax.experimental.pallas import tpu as pltpu

_EPS = 1e-08
_BLOCK = 512


def _route_block(hb, a_n):
    h_n = hb / jnp.maximum(
        jnp.sqrt(jnp.sum(hb * hb, axis=1, keepdims=True)), _EPS)
    scores = jax.lax.dot_general(
        h_n, a_n, (((1,), (1,)), ((), ())),
        preferred_element_type=jnp.float32)
    return jnp.argmax(scores, axis=1).astype(jnp.int32)


def _router_kernel(h0_ref, h1_ref, h2_ref, h3_ref, a_ref,
                   o0_ref, o1_ref, o2_ref, o3_ref):
    a = a_ref[...]
    a_n = a / jnp.maximum(
        jnp.sqrt(jnp.sum(a * a, axis=1, keepdims=True)), _EPS)
    o0_ref[...] = _route_block(h0_ref[...], a_n)[None, None, :]
    o1_ref[...] = _route_block(h1_ref[...], a_n)[None, None, :]
    o2_ref[...] = _route_block(h2_ref[...], a_n)[None, None, :]
    o3_ref[...] = _route_block(h3_ref[...], a_n)[None, None, :]


def kernel(h, A):
    B, D = h.shape
    K = A.shape[0]
    nblk = B // _BLOCK
    nstep = nblk // 4
    o0, o1, o2, o3 = pl.pallas_call(
        _router_kernel,
        grid=(nstep,),
        in_specs=[
            pl.BlockSpec((_BLOCK, D), lambda i: (4 * i, 0)),
            pl.BlockSpec((_BLOCK, D), lambda i: (4 * i + 1, 0)),
            pl.BlockSpec((_BLOCK, D), lambda i: (4 * i + 2, 0)),
            pl.BlockSpec((_BLOCK, D), lambda i: (4 * i + 3, 0)),
            pl.BlockSpec((K, D), lambda i: (0, 0)),
        ],
        out_specs=[
            pl.BlockSpec((1, 1, _BLOCK), lambda i: (i, 0, 0)),
            pl.BlockSpec((1, 1, _BLOCK), lambda i: (i, 0, 0)),
            pl.BlockSpec((1, 1, _BLOCK), lambda i: (i, 0, 0)),
            pl.BlockSpec((1, 1, _BLOCK), lambda i: (i, 0, 0)),
        ],
        out_shape=[
            jax.ShapeDtypeStruct((nstep, 1, _BLOCK), jnp.int32),
            jax.ShapeDtypeStruct((nstep, 1, _BLOCK), jnp.int32),
            jax.ShapeDtypeStruct((nstep, 1, _BLOCK), jnp.int32),
            jax.ShapeDtypeStruct((nstep, 1, _BLOCK), jnp.int32),
        ],
        compiler_params=pltpu.CompilerParams(
            dimension_semantics=("arbitrary",),
            vmem_limit_bytes=100 * 1024 * 1024,
        ),
    )(h, h, h, h, A)
    idx = jnp.concatenate([o0, o1, o2, o3], axis=1).reshape(B, 1)
    return idx
